# baseline (speedup 1.0000x reference)
import functools

import jax
import jax.numpy as jnp
from jax import lax
from jax.experimental import pallas as pl
from jax.experimental.pallas import tpu as pltpu
from jax.experimental.pallas import tpu_sc as plsc

T = 2048
D = 2048
QL = 1536
H = 32
HD = 128
RD = 64
TOPK = 1024

TM = 256
HB = 8
NEG = float(jnp.finfo(jnp.float32).min)


def _rope_mats():
    a = lax.broadcasted_iota(jnp.int32, (RD, RD), 0)
    b = lax.broadcasted_iota(jnp.int32, (RD, RD), 1)
    half = RD // 2
    p = jnp.where((a % 2 == 0) & (b * 2 == a), 1.0, 0.0) + jnp.where(
        (a % 2 == 1) & (b == half + (a - 1) // 2), 1.0, 0.0)
    pr = jnp.where((a % 2 == 1) & (b * 2 + 1 == a), -1.0, 0.0) + jnp.where(
        (a % 2 == 0) & (b == half + a // 2), 1.0, 0.0)
    return p.astype(jnp.float32), pr.astype(jnp.float32)


def _qproj_kernel(ql_ref, wq_ref, cos_ref, sin_ref, q_ref):
    q = lax.dot_general(ql_ref[...], wq_ref[...],
                        (((1,), (1,)), ((), ())),
                        preferred_element_type=jnp.float32)
    p, pr = _rope_mats()
    cos = cos_ref[...]
    sin = sin_ref[...]
    parts = []
    for h in range(HB):
        pe = q[:, h * HD:h * HD + RD]
        xs = jnp.dot(pe, p, preferred_element_type=jnp.float32,
                     precision=lax.Precision.HIGHEST)
        rot = jnp.dot(pe, pr, preferred_element_type=jnp.float32,
                      precision=lax.Precision.HIGHEST)
        parts.append(xs * cos + rot * sin)
        parts.append(q[:, h * HD + RD:(h + 1) * HD])
    q_ref[...] = jnp.concatenate(parts, axis=1)


def _kw_kernel(x_ref, wk_ref, ww_ref, lnw_ref, lnb_ref, cos_ref, sin_ref,
               k_ref, w_ref):
    x = x_ref[...]
    kk = lax.dot_general(x, wk_ref[...], (((1,), (1,)), ((), ())),
                         preferred_element_type=jnp.float32)
    mu = jnp.mean(kk, axis=-1, keepdims=True)
    var = jnp.mean(jnp.square(kk - mu), axis=-1, keepdims=True)
    kk = (kk - mu) / jnp.sqrt(var + 1e-06) * lnw_ref[...] + lnb_ref[...]
    p, pr = _rope_mats()
    pe = kk[:, :RD]
    xs = jnp.dot(pe, p, preferred_element_type=jnp.float32,
                 precision=lax.Precision.HIGHEST)
    rot = jnp.dot(pe, pr, preferred_element_type=jnp.float32,
                  precision=lax.Precision.HIGHEST)
    roped = xs * cos_ref[...] + rot * sin_ref[...]
    k_ref[...] = jnp.concatenate([roped, kk[:, RD:]], axis=1)
    w_ref[...] = lax.dot_general(x, ww_ref[...], (((1,), (1,)), ((), ())),
                                 preferred_element_type=jnp.float32)


def _scores_kernel(q_ref, k_ref, w_ref, ks_ref, ke_ref, out_ref):
    scale = HD ** (-0.5) * H ** (-0.5)
    w = (w_ref[...] * scale).astype(jnp.bfloat16).astype(jnp.float32)
    k = k_ref[...]
    prods = []
    for h in range(H):
        qh = q_ref[:, h * HD:(h + 1) * HD]
        logits = lax.dot_general(qh, k, (((1,), (1,)), ((), ())),
                                 preferred_element_type=jnp.float32)
        rl = jnp.maximum(logits, 0.0).astype(jnp.bfloat16).astype(jnp.float32)
        prods.append(w[:, h:h + 1] * rl)
    acc = None
    for g in range(0, H, 8):
        ps = prods[g:g + 8]
        while len(ps) > 1:
            ps = [ps[i] + ps[i + 1] for i in range(0, len(ps), 2)]
        acc = ps[0] if acc is None else acc + ps[0]
    pos = lax.broadcasted_iota(jnp.int32, (TM, T), 1)
    valid = (pos >= ks_ref[0]) & (pos < ke_ref[0])
    ui = lax.bitcast_convert_type(acc, jnp.uint32)
    su = jnp.where(acc < 0.0, ~ui, ui | jnp.uint32(0x80000000))
    key = ~su
    key = jnp.where(valid, key, jnp.uint32(0xFFFFFFFF))
    out_ref[...] = lax.bitcast_convert_type(key, jnp.int32)


def _layernorm_host(x, w, b, eps=1e-06):
    mu = jnp.mean(x, axis=-1, keepdims=True)
    var = jnp.mean(jnp.square(x - mu), axis=-1, keepdims=True)
    return (x - mu) / jnp.sqrt(var + eps) * w + b


def _rope_interleave_host(x, cos, sin):
    d = x.shape[-1]
    xs = x.reshape(x.shape[:-1] + (d // 2, 2))
    xs = jnp.swapaxes(xs, -1, -2).reshape(x.shape)
    rot = jnp.concatenate([-xs[..., d // 2:], xs[..., :d // 2]], axis=-1)
    return xs * cos + rot * sin


def _scores(hidden_states, q_latent, ks, ke, cos, sin, Wq_b, Wk, ln_w, ln_b,
            Ww):
    x = hidden_states[0]
    ql = q_latent[0]
    cos_t = cos[0]
    sin_t = sin[0]

    nt = T // TM
    q_idx = (ql @ Wq_b.T).reshape(T, H, HD)
    k_idx = _layernorm_host(x @ Wk.T, ln_w, ln_b)
    w = x @ Ww.T
    q_pe = _rope_interleave_host(q_idx[..., :RD], cos_t[:, None, :],
                                 sin_t[:, None, :])
    k_pe = _rope_interleave_host(k_idx[:, :RD], cos_t, sin_t)
    q = jnp.concatenate([q_pe, q_idx[..., RD:]], -1).reshape(T, H * HD)
    k = jnp.concatenate([k_pe, k_idx[:, RD:]], -1)

    ks3 = ks.reshape(nt, TM, 1)
    ke3 = ke.reshape(nt, TM, 1)
    masked = pl.pallas_call(
        _scores_kernel,
        grid=(nt,),
        in_specs=[
            pl.BlockSpec((TM, H * HD), lambda i: (i, 0)),
            pl.BlockSpec((T, HD), lambda i: (0, 0)),
            pl.BlockSpec((TM, H), lambda i: (i, 0)),
            pl.BlockSpec((1, TM, 1), lambda i: (i, 0, 0)),
            pl.BlockSpec((1, TM, 1), lambda i: (i, 0, 0)),
        ],
        out_specs=pl.BlockSpec((TM, T), lambda i: (i, 0)),
        out_shape=jax.ShapeDtypeStruct((T, T), jnp.int32),
    )(q, k, w, ks3, ke3)
    return masked


NW = 32
RPW = T // NW


def _sc_topk(keys):
    mesh = plsc.VectorSubcoreMesh(core_axis_name="c", subcore_axis_name="s")
    NV = T // 16

    @functools.partial(
        pl.kernel,
        out_type=jax.ShapeDtypeStruct((T, TOPK), jnp.int32),
        mesh=mesh,
        scratch_types=[
            pltpu.VMEM((T,), jnp.int32),
            pltpu.VMEM((T,), jnp.int32),
            pltpu.VMEM((T,), jnp.int32),
            pltpu.VMEM((T,), jnp.int32),
            pltpu.VMEM((512,), jnp.int32),
            pltpu.VMEM((512,), jnp.int32),
            pltpu.VMEM((TOPK,), jnp.int32),
        ],
        compiler_params=pltpu.CompilerParams(needs_layout_passes=False),
    )
    def body(keys_hbm, out_hbm, kb0, vb0, kb1, vb1, hist, offs, outv):
        wid = lax.axis_index("s") * 2 + lax.axis_index("c")
        lane = jax.lax.iota(jnp.int32, 16)
        ones = jnp.ones((16,), jnp.int32)

        def do_pass(kb_in, vb_in, kb_out, vb_out, shift, first):
            def zbody(jj, c):
                hist[pl.ds(jj * 16, 16)] = jnp.zeros((16,), jnp.int32)
                return c
            lax.fori_loop(0, 32, zbody, 0)

            def hbody(i, c):
                idx = lane * NV + i
                k = plsc.load_gather(kb_in, [idx])
                digit = (k >> shift) & 31
                addr = digit * 16 + lane
                plsc.addupdate_scatter(hist, [addr], ones)
                return c
            lax.fori_loop(0, NV, hbody, 0)

            def obody(jj, carry):
                h = hist[pl.ds(jj * 16, 16)]
                cs = plsc.cumsum(h)
                offs[pl.ds(jj * 16, 16)] = cs - h + carry
                return carry + jnp.sum(h)
            lax.fori_loop(0, 32, obody, jnp.int32(0))

            def pbody(i, c):
                idx = lane * NV + i
                k = plsc.load_gather(kb_in, [idx])
                v = idx if first else plsc.load_gather(vb_in, [idx])
                digit = (k >> shift) & 31
                addr = digit * 16 + lane
                cur = plsc.load_gather(offs, [addr])
                plsc.store_scatter(kb_out, [cur], k)
                plsc.store_scatter(vb_out, [cur], v)
                plsc.store_scatter(offs, [addr], cur + ones)
                return c
            lax.fori_loop(0, NV, pbody, 0)

        def row_body(j, c):
            r = wid * RPW + j
            pltpu.sync_copy(keys_hbm.at[r], kb0)
            do_pass(kb0, vb0, kb1, vb1, 0, True)
            do_pass(kb1, vb1, kb0, vb0, 5, False)
            do_pass(kb0, vb0, kb1, vb1, 10, False)
            do_pass(kb1, vb1, kb0, vb0, 15, False)
            do_pass(kb0, vb0, kb1, vb1, 20, False)
            do_pass(kb1, vb1, kb0, vb0, 25, False)
            do_pass(kb0, vb0, kb1, vb1, 30, False)

            def wbody(jo, c2):
                key = kb1[pl.ds(jo * 16, 16)]
                val = vb1[pl.ds(jo * 16, 16)]
                outv[pl.ds(jo * 16, 16)] = jnp.where(key != -1, val, -1)
                return c2
            lax.fori_loop(0, TOPK // 16, wbody, 0)
            pltpu.sync_copy(outv, out_hbm.at[r])
            return c
        lax.fori_loop(0, RPW, row_body, 0)

    return body(keys)


def kernel(hidden_states, q_latent, ks, ke, index_topk, cos, sin, Wq_b, Wk,
           ln_w, ln_b, Ww):
    keys = _scores(hidden_states, q_latent, ks, ke, cos, sin, Wq_b, Wk,
                   ln_w, ln_b, Ww)
    top_idx = _sc_topk(keys)
    indices = jnp.where(jnp.arange(TOPK)[None, :] < index_topk, top_idx, -1)
    return indices.reshape(1, T, 1, TOPK)


# speedup vs baseline: 3.1087x; 3.1087x over previous
import functools

import jax
import jax.numpy as jnp
from jax import lax
from jax.experimental import pallas as pl
from jax.experimental.pallas import tpu as pltpu
from jax.experimental.pallas import tpu_sc as plsc

T = 2048
D = 2048
QL = 1536
H = 32
HD = 128
RD = 64
TOPK = 1024

TM = 256
HB = 8
NEG = float(jnp.finfo(jnp.float32).min)


def _rope_mats():
    a = lax.broadcasted_iota(jnp.int32, (RD, RD), 0)
    b = lax.broadcasted_iota(jnp.int32, (RD, RD), 1)
    half = RD // 2
    p = jnp.where((a % 2 == 0) & (b * 2 == a), 1.0, 0.0) + jnp.where(
        (a % 2 == 1) & (b == half + (a - 1) // 2), 1.0, 0.0)
    pr = jnp.where((a % 2 == 1) & (b * 2 + 1 == a), -1.0, 0.0) + jnp.where(
        (a % 2 == 0) & (b == half + a // 2), 1.0, 0.0)
    return p.astype(jnp.float32), pr.astype(jnp.float32)


def _qproj_kernel(ql_ref, wq_ref, cos_ref, sin_ref, q_ref):
    q = lax.dot_general(ql_ref[...], wq_ref[...],
                        (((1,), (1,)), ((), ())),
                        preferred_element_type=jnp.float32)
    p, pr = _rope_mats()
    cos = cos_ref[...]
    sin = sin_ref[...]
    parts = []
    for h in range(HB):
        pe = q[:, h * HD:h * HD + RD]
        xs = jnp.dot(pe, p, preferred_element_type=jnp.float32,
                     precision=lax.Precision.HIGHEST)
        rot = jnp.dot(pe, pr, preferred_element_type=jnp.float32,
                      precision=lax.Precision.HIGHEST)
        parts.append(xs * cos + rot * sin)
        parts.append(q[:, h * HD + RD:(h + 1) * HD])
    q_ref[...] = jnp.concatenate(parts, axis=1)


def _kw_kernel(x_ref, wk_ref, ww_ref, lnw_ref, lnb_ref, cos_ref, sin_ref,
               k_ref, w_ref):
    x = x_ref[...]
    kk = lax.dot_general(x, wk_ref[...], (((1,), (1,)), ((), ())),
                         preferred_element_type=jnp.float32)
    mu = jnp.mean(kk, axis=-1, keepdims=True)
    var = jnp.mean(jnp.square(kk - mu), axis=-1, keepdims=True)
    kk = (kk - mu) / jnp.sqrt(var + 1e-06) * lnw_ref[...] + lnb_ref[...]
    p, pr = _rope_mats()
    pe = kk[:, :RD]
    xs = jnp.dot(pe, p, preferred_element_type=jnp.float32,
                 precision=lax.Precision.HIGHEST)
    rot = jnp.dot(pe, pr, preferred_element_type=jnp.float32,
                  precision=lax.Precision.HIGHEST)
    roped = xs * cos_ref[...] + rot * sin_ref[...]
    k_ref[...] = jnp.concatenate([roped, kk[:, RD:]], axis=1)
    w_ref[...] = lax.dot_general(x, ww_ref[...], (((1,), (1,)), ((), ())),
                                 preferred_element_type=jnp.float32)


def _scores_kernel(q_ref, k_ref, w_ref, ks_ref, ke_ref, out_ref):
    scale = HD ** (-0.5) * H ** (-0.5)
    w = (w_ref[...] * scale).astype(jnp.bfloat16).astype(jnp.float32)
    k = k_ref[...]
    prods = []
    for h in range(H):
        qh = q_ref[:, h * HD:(h + 1) * HD]
        logits = lax.dot_general(qh, k, (((1,), (1,)), ((), ())),
                                 preferred_element_type=jnp.float32)
        rl = jnp.maximum(logits, 0.0).astype(jnp.bfloat16).astype(jnp.float32)
        prods.append(w[:, h:h + 1] * rl)
    acc = None
    for g in range(0, H, 8):
        ps = prods[g:g + 8]
        while len(ps) > 1:
            ps = [ps[i] + ps[i + 1] for i in range(0, len(ps), 2)]
        acc = ps[0] if acc is None else acc + ps[0]
    pos = lax.broadcasted_iota(jnp.int32, (TM, T), 1)
    valid = (pos >= ks_ref[0]) & (pos < ke_ref[0])
    ui = lax.bitcast_convert_type(acc, jnp.uint32)
    su = jnp.where(acc < 0.0, ~ui, ui | jnp.uint32(0x80000000))
    key = ~su
    key = jnp.where(valid, key, jnp.uint32(0xFFFFFFFF))
    out_ref[...] = lax.bitcast_convert_type(key, jnp.int32)


def _layernorm_host(x, w, b, eps=1e-06):
    mu = jnp.mean(x, axis=-1, keepdims=True)
    var = jnp.mean(jnp.square(x - mu), axis=-1, keepdims=True)
    return (x - mu) / jnp.sqrt(var + eps) * w + b


def _rope_interleave_host(x, cos, sin):
    d = x.shape[-1]
    xs = x.reshape(x.shape[:-1] + (d // 2, 2))
    xs = jnp.swapaxes(xs, -1, -2).reshape(x.shape)
    rot = jnp.concatenate([-xs[..., d // 2:], xs[..., :d // 2]], axis=-1)
    return xs * cos + rot * sin


def _scores(hidden_states, q_latent, ks, ke, cos, sin, Wq_b, Wk, ln_w, ln_b,
            Ww):
    x = hidden_states[0]
    ql = q_latent[0]
    cos_t = cos[0]
    sin_t = sin[0]

    nt = T // TM
    q_idx = (ql @ Wq_b.T).reshape(T, H, HD)
    k_idx = _layernorm_host(x @ Wk.T, ln_w, ln_b)
    w = x @ Ww.T
    q_pe = _rope_interleave_host(q_idx[..., :RD], cos_t[:, None, :],
                                 sin_t[:, None, :])
    k_pe = _rope_interleave_host(k_idx[:, :RD], cos_t, sin_t)
    q = jnp.concatenate([q_pe, q_idx[..., RD:]], -1).reshape(T, H * HD)
    k = jnp.concatenate([k_pe, k_idx[:, RD:]], -1)

    ks3 = ks.reshape(nt, TM, 1)
    ke3 = ke.reshape(nt, TM, 1)
    masked = pl.pallas_call(
        _scores_kernel,
        grid=(nt,),
        in_specs=[
            pl.BlockSpec((TM, H * HD), lambda i: (i, 0)),
            pl.BlockSpec((T, HD), lambda i: (0, 0)),
            pl.BlockSpec((TM, H), lambda i: (i, 0)),
            pl.BlockSpec((1, TM, 1), lambda i: (i, 0, 0)),
            pl.BlockSpec((1, TM, 1), lambda i: (i, 0, 0)),
        ],
        out_specs=pl.BlockSpec((TM, T), lambda i: (i, 0)),
        out_shape=jax.ShapeDtypeStruct((T, T), jnp.int32),
    )(q, k, w, ks3, ke3)
    return masked


NW = 32
RPW = T // NW


def _sc_topk(keys, ke):
    mesh = plsc.VectorSubcoreMesh(core_axis_name="c", subcore_axis_name="s")
    NV = T // 16

    @functools.partial(
        pl.kernel,
        out_type=jax.ShapeDtypeStruct((T, TOPK), jnp.int32),
        mesh=mesh,
        scratch_types=[
            pltpu.VMEM((T,), jnp.int32),
            pltpu.VMEM((T,), jnp.int32),
            pltpu.VMEM((T,), jnp.int32),
            pltpu.VMEM((T,), jnp.int32),
            pltpu.VMEM((512,), jnp.int32),
            pltpu.VMEM((512,), jnp.int32),
            pltpu.VMEM((TOPK,), jnp.int32),
            pltpu.VMEM((T,), jnp.int32),
        ],
        compiler_params=pltpu.CompilerParams(needs_layout_passes=False),
    )
    def body(keys_hbm, ke_hbm, out_hbm, kb0, vb0, kb1, vb1, hist, offs, outv,
             kev):
        wid = lax.axis_index("s") * 2 + lax.axis_index("c")
        lane = jax.lax.iota(jnp.int32, 16)
        ones = jnp.ones((16,), jnp.int32)
        pltpu.sync_copy(ke_hbm, kev)

        def do_pass(cs, kb_in, vb_in, kb_out, vb_out, shift, first):
            def zbody(jj, c):
                hist[pl.ds(jj * 16, 16)] = jnp.zeros((16,), jnp.int32)
                return c
            lax.fori_loop(0, 32, zbody, 0)

            def hbody(i, c):
                idx = lane * cs + i
                k = plsc.load_gather(kb_in, [idx])
                digit = (k >> shift) & 31
                addr = digit * 16 + lane
                plsc.addupdate_scatter(hist, [addr], ones)
                return c
            lax.fori_loop(0, cs, hbody, 0)

            def obody(jj, carry):
                h = hist[pl.ds(jj * 16, 16)]
                csum = plsc.cumsum(h)
                offs[pl.ds(jj * 16, 16)] = csum - h + carry
                return carry + jnp.sum(h)
            lax.fori_loop(0, 32, obody, jnp.int32(0))

            def pbody(i, c):
                idx = lane * cs + i
                k = plsc.load_gather(kb_in, [idx])
                v = idx if first else plsc.load_gather(vb_in, [idx])
                digit = (k >> shift) & 31
                addr = digit * 16 + lane
                cur = plsc.load_gather(offs, [addr])
                plsc.store_scatter(kb_out, [cur], k)
                plsc.store_scatter(vb_out, [cur], v)
                plsc.store_scatter(offs, [addr], cur + ones)
                return c
            lax.fori_loop(0, cs, pbody, 0)

        def row_body(j, c):
            r = wid + NW * j
            wl = wid & 15
            kvec = kev[pl.ds(r - wl, 16)]
            n = jnp.sum(jnp.where(lane == wl, kvec, jnp.zeros((16,), jnp.int32)))
            cs = (n + 15) // 16
            pltpu.sync_copy(keys_hbm.at[r], kb0)
            do_pass(cs, kb0, vb0, kb1, vb1, 0, True)
            do_pass(cs, kb1, vb1, kb0, vb0, 5, False)
            do_pass(cs, kb0, vb0, kb1, vb1, 10, False)
            do_pass(cs, kb1, vb1, kb0, vb0, 15, False)
            do_pass(cs, kb0, vb0, kb1, vb1, 20, False)
            do_pass(cs, kb1, vb1, kb0, vb0, 25, False)
            do_pass(cs, kb0, vb0, kb1, vb1, 30, False)

            wlim = jnp.minimum(cs, TOPK // 16)

            def wbody(jo, c2):
                key = kb1[pl.ds(jo * 16, 16)]
                val = vb1[pl.ds(jo * 16, 16)]
                outv[pl.ds(jo * 16, 16)] = jnp.where(key != -1, val, -1)
                return c2
            lax.fori_loop(0, wlim, wbody, 0)

            def fbody(jo, c2):
                outv[pl.ds(jo * 16, 16)] = jnp.full((16,), -1, jnp.int32)
                return c2
            lax.fori_loop(wlim, TOPK // 16, fbody, 0)
            pltpu.sync_copy(outv, out_hbm.at[r])
            return c
        lax.fori_loop(0, RPW, row_body, 0)

    return body(keys, ke)


def kernel(hidden_states, q_latent, ks, ke, index_topk, cos, sin, Wq_b, Wk,
           ln_w, ln_b, Ww):
    keys = _scores(hidden_states, q_latent, ks, ke, cos, sin, Wq_b, Wk,
                   ln_w, ln_b, Ww)
    top_idx = _sc_topk(keys, ke)
    indices = jnp.where(jnp.arange(TOPK)[None, :] < index_topk, top_idx, -1)
    return indices.reshape(1, T, 1, TOPK)


# speedup vs baseline: 3.6223x; 1.1652x over previous
import functools

import jax
import jax.numpy as jnp
from jax import lax
from jax.experimental import pallas as pl
from jax.experimental.pallas import tpu as pltpu
from jax.experimental.pallas import tpu_sc as plsc

T = 2048
D = 2048
QL = 1536
H = 32
HD = 128
RD = 64
TOPK = 1024

TM = 256
HB = 8
NEG = float(jnp.finfo(jnp.float32).min)


def _rope_mats():
    a = lax.broadcasted_iota(jnp.int32, (RD, RD), 0)
    b = lax.broadcasted_iota(jnp.int32, (RD, RD), 1)
    half = RD // 2
    p = jnp.where((a % 2 == 0) & (b * 2 == a), 1.0, 0.0) + jnp.where(
        (a % 2 == 1) & (b == half + (a - 1) // 2), 1.0, 0.0)
    pr = jnp.where((a % 2 == 1) & (b * 2 + 1 == a), -1.0, 0.0) + jnp.where(
        (a % 2 == 0) & (b == half + a // 2), 1.0, 0.0)
    return p.astype(jnp.float32), pr.astype(jnp.float32)


def _qproj_kernel(ql_ref, wq_ref, cos_ref, sin_ref, q_ref):
    q = lax.dot_general(ql_ref[...], wq_ref[...],
                        (((1,), (1,)), ((), ())),
                        preferred_element_type=jnp.float32)
    p, pr = _rope_mats()
    cos = cos_ref[...]
    sin = sin_ref[...]
    parts = []
    for h in range(HB):
        pe = q[:, h * HD:h * HD + RD]
        xs = jnp.dot(pe, p, preferred_element_type=jnp.float32,
                     precision=lax.Precision.HIGHEST)
        rot = jnp.dot(pe, pr, preferred_element_type=jnp.float32,
                      precision=lax.Precision.HIGHEST)
        parts.append(xs * cos + rot * sin)
        parts.append(q[:, h * HD + RD:(h + 1) * HD])
    q_ref[...] = jnp.concatenate(parts, axis=1)


def _kw_kernel(x_ref, wk_ref, ww_ref, lnw_ref, lnb_ref, cos_ref, sin_ref,
               k_ref, w_ref):
    x = x_ref[...]
    kk = lax.dot_general(x, wk_ref[...], (((1,), (1,)), ((), ())),
                         preferred_element_type=jnp.float32)
    mu = jnp.mean(kk, axis=-1, keepdims=True)
    var = jnp.mean(jnp.square(kk - mu), axis=-1, keepdims=True)
    kk = (kk - mu) / jnp.sqrt(var + 1e-06) * lnw_ref[...] + lnb_ref[...]
    p, pr = _rope_mats()
    pe = kk[:, :RD]
    xs = jnp.dot(pe, p, preferred_element_type=jnp.float32,
                 precision=lax.Precision.HIGHEST)
    rot = jnp.dot(pe, pr, preferred_element_type=jnp.float32,
                  precision=lax.Precision.HIGHEST)
    roped = xs * cos_ref[...] + rot * sin_ref[...]
    k_ref[...] = jnp.concatenate([roped, kk[:, RD:]], axis=1)
    w_ref[...] = lax.dot_general(x, ww_ref[...], (((1,), (1,)), ((), ())),
                                 preferred_element_type=jnp.float32)


def _scores_kernel(q_ref, k_ref, w_ref, ks_ref, ke_ref, out_ref):
    scale = HD ** (-0.5) * H ** (-0.5)
    w = (w_ref[...] * scale).astype(jnp.bfloat16).astype(jnp.float32)
    k = k_ref[...]
    prods = []
    for h in range(H):
        qh = q_ref[:, h * HD:(h + 1) * HD]
        logits = lax.dot_general(qh, k, (((1,), (1,)), ((), ())),
                                 preferred_element_type=jnp.float32)
        rl = jnp.maximum(logits, 0.0).astype(jnp.bfloat16).astype(jnp.float32)
        prods.append(w[:, h:h + 1] * rl)
    acc = None
    for g in range(0, H, 8):
        ps = prods[g:g + 8]
        while len(ps) > 1:
            ps = [ps[i] + ps[i + 1] for i in range(0, len(ps), 2)]
        acc = ps[0] if acc is None else acc + ps[0]
    pos = lax.broadcasted_iota(jnp.int32, (TM, T), 1)
    valid = (pos >= ks_ref[0]) & (pos < ke_ref[0])
    ui = lax.bitcast_convert_type(acc, jnp.uint32)
    su = jnp.where(acc < 0.0, ~ui, ui | jnp.uint32(0x80000000))
    key = ~su
    key = jnp.where(valid, key, jnp.uint32(0xFFFFFFFF))
    out_ref[...] = lax.bitcast_convert_type(key, jnp.int32)


def _layernorm_host(x, w, b, eps=1e-06):
    mu = jnp.mean(x, axis=-1, keepdims=True)
    var = jnp.mean(jnp.square(x - mu), axis=-1, keepdims=True)
    return (x - mu) / jnp.sqrt(var + eps) * w + b


def _rope_interleave_host(x, cos, sin):
    d = x.shape[-1]
    xs = x.reshape(x.shape[:-1] + (d // 2, 2))
    xs = jnp.swapaxes(xs, -1, -2).reshape(x.shape)
    rot = jnp.concatenate([-xs[..., d // 2:], xs[..., :d // 2]], axis=-1)
    return xs * cos + rot * sin


def _scores(hidden_states, q_latent, ks, ke, cos, sin, Wq_b, Wk, ln_w, ln_b,
            Ww):
    x = hidden_states[0]
    ql = q_latent[0]
    cos_t = cos[0]
    sin_t = sin[0]

    nt = T // TM
    q_idx = (ql @ Wq_b.T).reshape(T, H, HD)
    k_idx = _layernorm_host(x @ Wk.T, ln_w, ln_b)
    w = x @ Ww.T
    q_pe = _rope_interleave_host(q_idx[..., :RD], cos_t[:, None, :],
                                 sin_t[:, None, :])
    k_pe = _rope_interleave_host(k_idx[:, :RD], cos_t, sin_t)
    q = jnp.concatenate([q_pe, q_idx[..., RD:]], -1).reshape(T, H * HD)
    k = jnp.concatenate([k_pe, k_idx[:, RD:]], -1)

    ks3 = ks.reshape(nt, TM, 1)
    ke3 = ke.reshape(nt, TM, 1)
    masked = pl.pallas_call(
        _scores_kernel,
        grid=(nt,),
        in_specs=[
            pl.BlockSpec((TM, H * HD), lambda i: (i, 0)),
            pl.BlockSpec((T, HD), lambda i: (0, 0)),
            pl.BlockSpec((TM, H), lambda i: (i, 0)),
            pl.BlockSpec((1, TM, 1), lambda i: (i, 0, 0)),
            pl.BlockSpec((1, TM, 1), lambda i: (i, 0, 0)),
        ],
        out_specs=pl.BlockSpec((TM, T), lambda i: (i, 0)),
        out_shape=jax.ShapeDtypeStruct((T, T), jnp.int32),
    )(q, k, w, ks3, ke3)
    return masked


NW = 32
RPW = T // NW


def _sc_topk(keys, ke):
    mesh = plsc.VectorSubcoreMesh(core_axis_name="c", subcore_axis_name="s")
    NV = T // 16

    @functools.partial(
        pl.kernel,
        out_type=jax.ShapeDtypeStruct((T, TOPK), jnp.int32),
        mesh=mesh,
        scratch_types=[
            pltpu.VMEM((T,), jnp.int32),
            pltpu.VMEM((T,), jnp.int32),
            pltpu.VMEM((T,), jnp.int32),
            pltpu.VMEM((T,), jnp.int32),
            pltpu.VMEM((512,), jnp.int32),
            pltpu.VMEM((512,), jnp.int32),
            pltpu.VMEM((T,), jnp.int32),
            pltpu.VMEM((T,), jnp.int32),
            pltpu.VMEM((T,), jnp.int32),
            pltpu.VMEM((T,), jnp.int32),
            pltpu.VMEM((512,), jnp.int32),
            pltpu.VMEM((512,), jnp.int32),
            pltpu.VMEM((TOPK,), jnp.int32),
            pltpu.VMEM((T,), jnp.int32),
        ],
        compiler_params=pltpu.CompilerParams(needs_layout_passes=False),
    )
    def body(keys_hbm, ke_hbm, out_hbm, ka0, va0, ka1, va1, hista, offsa,
             kb0, vb0, kb1, vb1, histb, offsb, outv, kev):
        wid = lax.axis_index("s") * 2 + lax.axis_index("c")
        lane = jax.lax.iota(jnp.int32, 16)
        ones = jnp.ones((16,), jnp.int32)
        zeros = jnp.zeros((16,), jnp.int32)
        pltpu.sync_copy(ke_hbm, kev)

        def do_pass(cs, ins, outs, shift, first):
            (kia, via, kib, vib) = ins
            (koa, voa, kob, vob) = outs

            def zbody(jj, c):
                hista[pl.ds(jj * 16, 16)] = zeros
                histb[pl.ds(jj * 16, 16)] = zeros
                return c
            lax.fori_loop(0, 32, zbody, 0)

            def hbody(i, c):
                idx = lane * cs + i
                ka = plsc.load_gather(kia, [idx])
                kb = plsc.load_gather(kib, [idx])
                addra = ((ka >> shift) & 31) * 16 + lane
                addrb = ((kb >> shift) & 31) * 16 + lane
                plsc.addupdate_scatter(hista, [addra], ones)
                plsc.addupdate_scatter(histb, [addrb], ones)
                return c
            lax.fori_loop(0, cs, hbody, 0)

            def obody(jj, carry):
                ca, cb = carry
                ha = hista[pl.ds(jj * 16, 16)]
                hb = histb[pl.ds(jj * 16, 16)]
                csa = plsc.cumsum(ha)
                csb = plsc.cumsum(hb)
                offsa[pl.ds(jj * 16, 16)] = csa - ha + ca
                offsb[pl.ds(jj * 16, 16)] = csb - hb + cb
                return (ca + jnp.sum(ha), cb + jnp.sum(hb))
            lax.fori_loop(0, 32, obody, (jnp.int32(0), jnp.int32(0)))

            def pbody(i, c):
                idx = lane * cs + i
                ka = plsc.load_gather(kia, [idx])
                kb = plsc.load_gather(kib, [idx])
                va = idx if first else plsc.load_gather(via, [idx])
                vb = idx if first else plsc.load_gather(vib, [idx])
                addra = ((ka >> shift) & 31) * 16 + lane
                addrb = ((kb >> shift) & 31) * 16 + lane
                cura = plsc.load_gather(offsa, [addra])
                curb = plsc.load_gather(offsb, [addrb])
                plsc.store_scatter(koa, [cura], ka)
                plsc.store_scatter(kob, [curb], kb)
                plsc.store_scatter(voa, [cura], va)
                plsc.store_scatter(vob, [curb], vb)
                plsc.store_scatter(offsa, [addra], cura + ones)
                plsc.store_scatter(offsb, [addrb], curb + ones)
                return c
            lax.fori_loop(0, cs, pbody, 0)

        def emit_row(r, kfin, vfin, cs):
            wlim = jnp.minimum(cs, TOPK // 16)

            def wbody(jo, c2):
                key = kfin[pl.ds(jo * 16, 16)]
                val = vfin[pl.ds(jo * 16, 16)]
                outv[pl.ds(jo * 16, 16)] = jnp.where(key != -1, val, -1)
                return c2
            lax.fori_loop(0, wlim, wbody, 0)

            def fbody(jo, c2):
                outv[pl.ds(jo * 16, 16)] = jnp.full((16,), -1, jnp.int32)
                return c2
            lax.fori_loop(wlim, TOPK // 16, fbody, 0)
            pltpu.sync_copy(outv, out_hbm.at[r])

        def row_body(jp, c):
            ra = wid + NW * (2 * jp)
            rb = wid + NW * (2 * jp + 1)
            wl = wid & 15
            kva = kev[pl.ds(ra - wl, 16)]
            kvb = kev[pl.ds(rb - wl, 16)]
            na = jnp.sum(jnp.where(lane == wl, kva, zeros))
            nb = jnp.sum(jnp.where(lane == wl, kvb, zeros))
            csa = (na + 15) // 16
            csb = (nb + 15) // 16
            cs = jnp.maximum(csa, csb)
            pltpu.sync_copy(keys_hbm.at[ra], ka0)
            pltpu.sync_copy(keys_hbm.at[rb], kb0)
            b0 = (ka0, va0, kb0, vb0)
            b1 = (ka1, va1, kb1, vb1)
            do_pass(cs, b0, b1, 0, True)
            do_pass(cs, b1, b0, 5, False)
            do_pass(cs, b0, b1, 10, False)
            do_pass(cs, b1, b0, 15, False)
            do_pass(cs, b0, b1, 20, False)
            do_pass(cs, b1, b0, 25, False)
            do_pass(cs, b0, b1, 30, False)
            emit_row(ra, ka1, va1, jnp.minimum(csa, cs))
            emit_row(rb, kb1, vb1, jnp.minimum(csb, cs))
            return c
        lax.fori_loop(0, RPW // 2, row_body, 0)

    return body(keys, ke)


def kernel(hidden_states, q_latent, ks, ke, index_topk, cos, sin, Wq_b, Wk,
           ln_w, ln_b, Ww):
    keys = _scores(hidden_states, q_latent, ks, ke, cos, sin, Wq_b, Wk,
                   ln_w, ln_b, Ww)
    top_idx = _sc_topk(keys, ke)
    indices = jnp.where(jnp.arange(TOPK)[None, :] < index_topk, top_idx, -1)
    return indices.reshape(1, T, 1, TOPK)


# speedup vs baseline: 3.6231x; 1.0002x over previous
import functools

import jax
import jax.numpy as jnp
from jax import lax
from jax.experimental import pallas as pl
from jax.experimental.pallas import tpu as pltpu
from jax.experimental.pallas import tpu_sc as plsc

T = 2048
D = 2048
QL = 1536
H = 32
HD = 128
RD = 64
TOPK = 1024

TM = 256


def _scores_kernel(q_ref, k_ref, w_ref, ks_ref, ke_ref, out_ref):
    scale = HD ** (-0.5) * H ** (-0.5)
    w = (w_ref[...] * scale).astype(jnp.bfloat16).astype(jnp.float32)
    k = k_ref[...]
    prods = []
    for h in range(H):
        qh = q_ref[:, h * HD:(h + 1) * HD]
        logits = lax.dot_general(qh, k, (((1,), (1,)), ((), ())),
                                 preferred_element_type=jnp.float32)
        rl = jnp.maximum(logits, 0.0).astype(jnp.bfloat16).astype(jnp.float32)
        prods.append(w[:, h:h + 1] * rl)
    acc = None
    for g in range(0, H, 8):
        ps = prods[g:g + 8]
        while len(ps) > 1:
            ps = [ps[i] + ps[i + 1] for i in range(0, len(ps), 2)]
        acc = ps[0] if acc is None else acc + ps[0]
    pos = lax.broadcasted_iota(jnp.int32, (TM, T), 1)
    valid = (pos >= ks_ref[0]) & (pos < ke_ref[0])
    ui = lax.bitcast_convert_type(acc, jnp.uint32)
    su = jnp.where(acc < 0.0, ~ui, ui | jnp.uint32(0x80000000))
    key = ~su
    key = jnp.where(valid, key, jnp.uint32(0xFFFFFFFF))
    out_ref[...] = lax.bitcast_convert_type(key, jnp.int32)


def _layernorm_host(x, w, b, eps=1e-06):
    mu = jnp.mean(x, axis=-1, keepdims=True)
    var = jnp.mean(jnp.square(x - mu), axis=-1, keepdims=True)
    return (x - mu) / jnp.sqrt(var + eps) * w + b


def _rope_interleave_host(x, cos, sin):
    d = x.shape[-1]
    xs = x.reshape(x.shape[:-1] + (d // 2, 2))
    xs = jnp.swapaxes(xs, -1, -2).reshape(x.shape)
    rot = jnp.concatenate([-xs[..., d // 2:], xs[..., :d // 2]], axis=-1)
    return xs * cos + rot * sin


def _scores(hidden_states, q_latent, ks, ke, cos, sin, Wq_b, Wk, ln_w, ln_b,
            Ww):
    x = hidden_states[0]
    ql = q_latent[0]
    cos_t = cos[0]
    sin_t = sin[0]

    nt = T // TM
    q_idx = (ql @ Wq_b.T).reshape(T, H, HD)
    k_idx = _layernorm_host(x @ Wk.T, ln_w, ln_b)
    w = x @ Ww.T
    q_pe = _rope_interleave_host(q_idx[..., :RD], cos_t[:, None, :],
                                 sin_t[:, None, :])
    k_pe = _rope_interleave_host(k_idx[:, :RD], cos_t, sin_t)
    q = jnp.concatenate([q_pe, q_idx[..., RD:]], -1).reshape(T, H * HD)
    k = jnp.concatenate([k_pe, k_idx[:, RD:]], -1)

    ks3 = ks.reshape(nt, TM, 1)
    ke3 = ke.reshape(nt, TM, 1)
    masked = pl.pallas_call(
        _scores_kernel,
        grid=(nt,),
        in_specs=[
            pl.BlockSpec((TM, H * HD), lambda i: (i, 0)),
            pl.BlockSpec((T, HD), lambda i: (0, 0)),
            pl.BlockSpec((TM, H), lambda i: (i, 0)),
            pl.BlockSpec((1, TM, 1), lambda i: (i, 0, 0)),
            pl.BlockSpec((1, TM, 1), lambda i: (i, 0, 0)),
        ],
        out_specs=pl.BlockSpec((TM, T), lambda i: (i, 0)),
        out_shape=jax.ShapeDtypeStruct((T, T), jnp.int32),
    )(q, k, w, ks3, ke3)
    return masked


NW = 32
RPW = T // NW


def _sc_topk(keys, ke):
    mesh = plsc.VectorSubcoreMesh(core_axis_name="c", subcore_axis_name="s")
    NV = T // 16

    @functools.partial(
        pl.kernel,
        out_type=jax.ShapeDtypeStruct((T, TOPK), jnp.int32),
        mesh=mesh,
        scratch_types=[
            pltpu.VMEM((T,), jnp.int32),
            pltpu.VMEM((T,), jnp.int32),
            pltpu.VMEM((T,), jnp.int32),
            pltpu.VMEM((T,), jnp.int32),
            pltpu.VMEM((512,), jnp.int32),
            pltpu.VMEM((512,), jnp.int32),
            pltpu.VMEM((T,), jnp.int32),
            pltpu.VMEM((T,), jnp.int32),
            pltpu.VMEM((T,), jnp.int32),
            pltpu.VMEM((T,), jnp.int32),
            pltpu.VMEM((512,), jnp.int32),
            pltpu.VMEM((512,), jnp.int32),
            pltpu.VMEM((TOPK,), jnp.int32),
            pltpu.VMEM((T,), jnp.int32),
        ],
        compiler_params=pltpu.CompilerParams(needs_layout_passes=False),
    )
    def body(keys_hbm, ke_hbm, out_hbm, ka0, va0, ka1, va1, hista, offsa,
             kb0, vb0, kb1, vb1, histb, offsb, outv, kev):
        wid = lax.axis_index("s") * 2 + lax.axis_index("c")
        lane = jax.lax.iota(jnp.int32, 16)
        ones = jnp.ones((16,), jnp.int32)
        zeros = jnp.zeros((16,), jnp.int32)
        pltpu.sync_copy(ke_hbm, kev)

        def do_pass(cs, ins, outs, shift, first):
            (kia, via, kib, vib) = ins
            (koa, voa, kob, vob) = outs

            def zbody(jj, c):
                hista[pl.ds(jj * 16, 16)] = zeros
                histb[pl.ds(jj * 16, 16)] = zeros
                return c
            lax.fori_loop(0, 32, zbody, 0)

            def hbody(i, c):
                idx = lane * cs + i
                ka = plsc.load_gather(kia, [idx])
                kb = plsc.load_gather(kib, [idx])
                addra = ((ka >> shift) & 31) * 16 + lane
                addrb = ((kb >> shift) & 31) * 16 + lane
                plsc.addupdate_scatter(hista, [addra], ones)
                plsc.addupdate_scatter(histb, [addrb], ones)
                return c
            lax.fori_loop(0, cs, hbody, 0)

            def obody(jj, carry):
                ca, cb = carry
                ha = hista[pl.ds(jj * 16, 16)]
                hb = histb[pl.ds(jj * 16, 16)]
                csa = plsc.cumsum(ha)
                csb = plsc.cumsum(hb)
                offsa[pl.ds(jj * 16, 16)] = csa - ha + ca
                offsb[pl.ds(jj * 16, 16)] = csb - hb + cb
                return (ca + jnp.sum(ha), cb + jnp.sum(hb))
            lax.fori_loop(0, 32, obody, (jnp.int32(0), jnp.int32(0)))

            def pbody(i, c):
                idx = lane * cs + i
                ka = plsc.load_gather(kia, [idx])
                kb = plsc.load_gather(kib, [idx])
                va = idx if first else plsc.load_gather(via, [idx])
                vb = idx if first else plsc.load_gather(vib, [idx])
                addra = ((ka >> shift) & 31) * 16 + lane
                addrb = ((kb >> shift) & 31) * 16 + lane
                cura = plsc.load_gather(offsa, [addra])
                curb = plsc.load_gather(offsb, [addrb])
                plsc.store_scatter(koa, [cura], ka)
                plsc.store_scatter(kob, [curb], kb)
                plsc.store_scatter(voa, [cura], va)
                plsc.store_scatter(vob, [curb], vb)
                plsc.store_scatter(offsa, [addra], cura + ones)
                plsc.store_scatter(offsb, [addrb], curb + ones)
                return c
            lax.fori_loop(0, cs, pbody, 0)

        def emit_row(r, kfin, vfin, cs):
            wlim = jnp.minimum(cs, TOPK // 16)

            def wbody(jo, c2):
                key = kfin[pl.ds(jo * 16, 16)]
                val = vfin[pl.ds(jo * 16, 16)]
                outv[pl.ds(jo * 16, 16)] = jnp.where(key != -1, val, -1)
                return c2
            lax.fori_loop(0, wlim, wbody, 0)

            def fbody(jo, c2):
                outv[pl.ds(jo * 16, 16)] = jnp.full((16,), -1, jnp.int32)
                return c2
            lax.fori_loop(wlim, TOPK // 16, fbody, 0)
            pltpu.sync_copy(outv, out_hbm.at[r])

        def row_body(jp, c):
            ra = wid + NW * (2 * jp)
            rb = wid + NW * (2 * jp + 1)
            wl = wid & 15
            kva = kev[pl.ds(ra - wl, 16)]
            kvb = kev[pl.ds(rb - wl, 16)]
            na = jnp.sum(jnp.where(lane == wl, kva, zeros))
            nb = jnp.sum(jnp.where(lane == wl, kvb, zeros))
            csa = (na + 15) // 16
            csb = (nb + 15) // 16
            cs = jnp.maximum(csa, csb)
            pltpu.sync_copy(keys_hbm.at[ra], ka0)
            pltpu.sync_copy(keys_hbm.at[rb], kb0)
            b0 = (ka0, va0, kb0, vb0)
            b1 = (ka1, va1, kb1, vb1)
            do_pass(cs, b0, b1, 0, True)
            do_pass(cs, b1, b0, 5, False)
            do_pass(cs, b0, b1, 10, False)
            do_pass(cs, b1, b0, 15, False)
            do_pass(cs, b0, b1, 20, False)
            do_pass(cs, b1, b0, 25, False)
            do_pass(cs, b0, b1, 30, False)
            emit_row(ra, ka1, va1, jnp.minimum(csa, cs))
            emit_row(rb, kb1, vb1, jnp.minimum(csb, cs))
            return c
        lax.fori_loop(0, RPW // 2, row_body, 0)

    return body(keys, ke)


def kernel(hidden_states, q_latent, ks, ke, index_topk, cos, sin, Wq_b, Wk,
           ln_w, ln_b, Ww):
    keys = _scores(hidden_states, q_latent, ks, ke, cos, sin, Wq_b, Wk,
                   ln_w, ln_b, Ww)
    top_idx = _sc_topk(keys, ke)
    indices = jnp.where(jnp.arange(TOPK)[None, :] < index_topk, top_idx, -1)
    return indices.reshape(1, T, 1, TOPK)
